# baseline (device time: 192999 ns/iter reference)
import jax
import jax.numpy as jnp
from jax import lax
from jax.experimental import pallas as pl
from jax.experimental.pallas import tpu as pltpu

BM = 384


def kernel(A, B):
    m, k = A.shape
    k2, n = B.shape
    assert k == k2
    nsteps = m // BM
    assert m % BM == 0

    def body(
        a_ref, b_hbm, out_ref, b_vmem, pbuf, sbuf, rbuf, ssbuf, srbuf,
        b_sem, send_sems, recv_sems, ssend_sems, srecv_sems,
    ):
        i = pl.program_id(0)
        my_x = lax.axis_index("x")
        my_y = lax.axis_index("y")
        partner = (1 - my_x, my_y)

        slot = lax.rem(i, 2)
        pslot = lax.rem(i + 1, 2)
        rslot = lax.rem(i, 3)
        prslot = lax.rem(i + 2, 3)

        @pl.when(i == 0)
        def _():
            barrier_sem = pltpu.get_barrier_semaphore()
            pl.semaphore_signal(
                barrier_sem, inc=1,
                device_id=partner, device_id_type=pl.DeviceIdType.MESH,
            )
            pl.semaphore_wait(barrier_sem, 1)
            cp = pltpu.make_async_copy(b_hbm, b_vmem, b_sem)
            cp.start()
            cp.wait()

        def data_desc(c_slot, c_rslot):
            return pltpu.make_async_remote_copy(
                src_ref=sbuf.at[c_slot],
                dst_ref=rbuf.at[c_rslot],
                send_sem=send_sems.at[c_slot],
                recv_sem=recv_sems.at[c_rslot],
                device_id=partner,
                device_id_type=pl.DeviceIdType.MESH,
            )

        def scale_desc(c_slot, c_rslot):
            return pltpu.make_async_remote_copy(
                src_ref=ssbuf.at[c_slot],
                dst_ref=srbuf.at[c_rslot],
                send_sem=ssend_sems.at[c_slot],
                recv_sem=srecv_sems.at[c_rslot],
                device_id=partner,
                device_id_type=pl.DeviceIdType.MESH,
            )

        @pl.when(i >= 2)
        def _():
            data_desc(slot, rslot).wait_send()
            scale_desc(slot, rslot).wait_send()

        @pl.when(i < nsteps)
        def _():
            p = jnp.dot(
                a_ref[...], b_vmem[...], preferred_element_type=jnp.float32
            )
            pbuf[slot] = p.astype(jnp.bfloat16)
            mx = jnp.max(jnp.abs(p), axis=1, keepdims=True)
            mx = jnp.maximum(mx, 1e-30)
            sbuf[slot] = jnp.round(p * (127.0 / mx)).astype(jnp.int8)
            ssbuf[slot] = mx * (1.0 / 127.0)

        @pl.when(i > 0)
        def _():
            scale_desc(pslot, prslot).wait_recv()
            data_desc(pslot, prslot).wait_recv()

        @pl.when(i < nsteps)
        def _():
            scale_desc(slot, rslot).start()
            data_desc(slot, rslot).start()

        @pl.when(i == nsteps)
        def _():
            data_desc(pslot, prslot).wait_send()
            scale_desc(pslot, prslot).wait_send()

        @pl.when(i > 0)
        def _():
            out_ref[...] = pbuf[pslot].astype(jnp.float32) + (
                rbuf[prslot].astype(jnp.float32) * srbuf[prslot]
            )

    call = pl.pallas_call(
        body,
        grid=(nsteps + 1,),
        out_shape=jax.ShapeDtypeStruct((m, n), jnp.float32),
        in_specs=[
            pl.BlockSpec((BM, k), lambda i: (jnp.minimum(i, nsteps - 1), 0)),
            pl.BlockSpec(memory_space=pl.ANY),
        ],
        out_specs=pl.BlockSpec((BM, n), lambda i: (jnp.maximum(i - 1, 0), 0)),
        scratch_shapes=[
            pltpu.VMEM((k, n), jnp.bfloat16),
            pltpu.VMEM((2, BM, n), jnp.bfloat16),
            pltpu.VMEM((2, BM, n), jnp.int8),
            pltpu.VMEM((3, BM, n), jnp.int8),
            pltpu.VMEM((2, BM, 1), jnp.float32),
            pltpu.VMEM((3, BM, 1), jnp.float32),
            pltpu.SemaphoreType.DMA,
            pltpu.SemaphoreType.DMA((2,)),
            pltpu.SemaphoreType.DMA((3,)),
            pltpu.SemaphoreType.DMA((2,)),
            pltpu.SemaphoreType.DMA((3,)),
        ],
        compiler_params=pltpu.CompilerParams(
            collective_id=0,
            vmem_limit_bytes=60 * 1024 * 1024,
        ),
    )
    return call(A.astype(jnp.bfloat16), B.astype(jnp.bfloat16))


# device time: 172309 ns/iter; 1.1201x vs baseline; 1.1201x over previous
import jax
import jax.numpy as jnp
from jax import lax
from jax.experimental import pallas as pl
from jax.experimental.pallas import tpu as pltpu

BM = 384


def kernel(A, B):
    m, k = A.shape
    k2, n = B.shape
    assert k == k2
    nsteps = m // BM
    assert m % BM == 0

    def body(
        a_ref, b_hbm, out_ref, b_vmem, pbuf, sbuf, rbuf, ssbuf, srbuf,
        b_sem, send_sems, recv_sems, ssend_sems, srecv_sems,
    ):
        i = pl.program_id(0)
        my_x = lax.axis_index("x")
        my_y = lax.axis_index("y")
        partner = (1 - my_x, my_y)

        slot = lax.rem(i, 2)
        pslot = lax.rem(i + 1, 2)
        rslot = lax.rem(i, 4)
        prslot = lax.rem(i + 3, 4)

        @pl.when(i == 0)
        def _():
            barrier_sem = pltpu.get_barrier_semaphore()
            pl.semaphore_signal(
                barrier_sem, inc=1,
                device_id=partner, device_id_type=pl.DeviceIdType.MESH,
            )
            pl.semaphore_wait(barrier_sem, 1)
            cp = pltpu.make_async_copy(b_hbm, b_vmem, b_sem)
            cp.start()
            cp.wait()

        def data_desc(c_slot, c_rslot):
            return pltpu.make_async_remote_copy(
                src_ref=sbuf.at[c_slot],
                dst_ref=rbuf.at[c_rslot],
                send_sem=send_sems.at[c_slot],
                recv_sem=recv_sems.at[c_rslot],
                device_id=partner,
                device_id_type=pl.DeviceIdType.MESH,
            )

        def scale_desc(c_slot, c_rslot):
            return pltpu.make_async_remote_copy(
                src_ref=ssbuf.at[c_slot],
                dst_ref=srbuf.at[c_rslot],
                send_sem=ssend_sems.at[c_slot],
                recv_sem=srecv_sems.at[c_rslot],
                device_id=partner,
                device_id_type=pl.DeviceIdType.MESH,
            )

        @pl.when(i >= 2)
        def _():
            data_desc(slot, rslot).wait_send()
            scale_desc(slot, rslot).wait_send()

        @pl.when(i < nsteps)
        def _():
            p = jnp.dot(
                a_ref[...].astype(jnp.bfloat16), b_vmem[...],
                preferred_element_type=jnp.float32,
            )
            pbuf[slot] = p.astype(jnp.bfloat16)
            mx = jnp.max(jnp.abs(p), axis=1, keepdims=True)
            mx = jnp.maximum(mx, 1e-30)
            sbuf[slot] = jnp.round(p * (127.0 / mx)).astype(jnp.int8)
            ssbuf[slot] = mx * (1.0 / 127.0)

        @pl.when(i < nsteps)
        def _():
            scale_desc(slot, rslot).start()
            data_desc(slot, rslot).start()

        @pl.when(i > 0)
        def _():
            scale_desc(pslot, prslot).wait_recv()
            data_desc(pslot, prslot).wait_recv()

        @pl.when(i == nsteps)
        def _():
            data_desc(pslot, prslot).wait_send()
            scale_desc(pslot, prslot).wait_send()

        @pl.when(i > 0)
        def _():
            out_ref[...] = pbuf[pslot].astype(jnp.float32) + (
                rbuf[prslot].astype(jnp.float32) * srbuf[prslot]
            )

    call = pl.pallas_call(
        body,
        grid=(nsteps + 1,),
        out_shape=jax.ShapeDtypeStruct((m, n), jnp.float32),
        in_specs=[
            pl.BlockSpec((BM, k), lambda i: (jnp.minimum(i, nsteps - 1), 0)),
            pl.BlockSpec(memory_space=pl.ANY),
        ],
        out_specs=pl.BlockSpec((BM, n), lambda i: (jnp.maximum(i - 1, 0), 0)),
        scratch_shapes=[
            pltpu.VMEM((k, n), jnp.bfloat16),
            pltpu.VMEM((2, BM, n), jnp.bfloat16),
            pltpu.VMEM((2, BM, n), jnp.int8),
            pltpu.VMEM((4, BM, n), jnp.int8),
            pltpu.VMEM((2, BM, 1), jnp.float32),
            pltpu.VMEM((4, BM, 1), jnp.float32),
            pltpu.SemaphoreType.DMA,
            pltpu.SemaphoreType.DMA((2,)),
            pltpu.SemaphoreType.DMA((4,)),
            pltpu.SemaphoreType.DMA((2,)),
            pltpu.SemaphoreType.DMA((4,)),
        ],
        compiler_params=pltpu.CompilerParams(
            collective_id=0,
            vmem_limit_bytes=60 * 1024 * 1024,
        ),
    )
    return call(A, B.astype(jnp.bfloat16))


# device time: 154827 ns/iter; 1.2465x vs baseline; 1.1129x over previous
import jax
import jax.numpy as jnp
from jax import lax
from jax.experimental import pallas as pl
from jax.experimental.pallas import tpu as pltpu

BM = 128


def kernel(A, B):
    m, k = A.shape
    k2, n = B.shape
    assert k == k2
    nsteps = m // BM
    assert m % BM == 0

    def body(
        a_ref, b_hbm, out_ref, b_vmem, pbuf, sbuf, rbuf,
        b_sem, send_sems, recv_sems,
    ):
        i = pl.program_id(0)
        my_x = lax.axis_index("x")
        my_y = lax.axis_index("y")
        partner = (1 - my_x, my_y)

        slot = lax.rem(i, 2)
        pslot = lax.rem(i + 1, 2)
        rslot = lax.rem(i, 4)
        prslot = lax.rem(i + 3, 4)

        @pl.when(i == 0)
        def _():
            barrier_sem = pltpu.get_barrier_semaphore()
            pl.semaphore_signal(
                barrier_sem, inc=1,
                device_id=partner, device_id_type=pl.DeviceIdType.MESH,
            )
            pl.semaphore_wait(barrier_sem, 1)
            cp = pltpu.make_async_copy(b_hbm, b_vmem, b_sem)
            cp.start()
            cp.wait()

        def data_desc(c_slot, c_rslot):
            return pltpu.make_async_remote_copy(
                src_ref=sbuf.at[c_slot],
                dst_ref=rbuf.at[c_rslot],
                send_sem=send_sems.at[c_slot],
                recv_sem=recv_sems.at[c_rslot],
                device_id=partner,
                device_id_type=pl.DeviceIdType.MESH,
            )

        @pl.when(i >= 2)
        def _():
            data_desc(slot, rslot).wait_send()

        @pl.when(i < nsteps)
        def _():
            p = jnp.dot(
                a_ref[...].astype(jnp.bfloat16), b_vmem[...],
                preferred_element_type=jnp.float32,
            )
            pbuf[slot] = p.astype(jnp.bfloat16)
            mx = jnp.max(jnp.abs(p), axis=1, keepdims=True)
            mx = jnp.maximum(mx, 1e-30)
            sbuf[slot, :, :n] = jnp.round(p * (127.0 / mx)).astype(jnp.int8)
            sval = mx * (1.0 / 127.0)
            e = jnp.floor(jnp.log2(sval))
            m_ = jnp.round(sval * jnp.exp2(-e) * 63.0)
            sbuf[slot, :, n:n + 1] = m_.astype(jnp.int8)
            sbuf[slot, :, n + 1:n + 2] = e.astype(jnp.int8)

        @pl.when(i < nsteps)
        def _():
            data_desc(slot, rslot).start()

        @pl.when(i > 0)
        def _():
            data_desc(pslot, prslot).wait_recv()

        @pl.when(i == nsteps)
        def _():
            data_desc(pslot, prslot).wait_send()

        @pl.when(i > 0)
        def _():
            rm = rbuf[prslot, :, n:n + 1].astype(jnp.float32)
            re = rbuf[prslot, :, n + 1:n + 2].astype(jnp.float32)
            rscale = rm * jnp.exp2(re) * (1.0 / 63.0)
            out_ref[...] = pbuf[pslot].astype(jnp.float32) + (
                rbuf[prslot, :, :n].astype(jnp.float32) * rscale
            )

    call = pl.pallas_call(
        body,
        grid=(nsteps + 1,),
        out_shape=jax.ShapeDtypeStruct((m, n), jnp.float32),
        in_specs=[
            pl.BlockSpec((BM, k), lambda i: (jnp.minimum(i, nsteps - 1), 0)),
            pl.BlockSpec(memory_space=pl.ANY),
        ],
        out_specs=pl.BlockSpec((BM, n), lambda i: (jnp.maximum(i - 1, 0), 0)),
        scratch_shapes=[
            pltpu.VMEM((k, n), jnp.bfloat16),
            pltpu.VMEM((2, BM, n), jnp.bfloat16),
            pltpu.VMEM((2, BM, n + 128), jnp.int8),
            pltpu.VMEM((4, BM, n + 128), jnp.int8),
            pltpu.SemaphoreType.DMA,
            pltpu.SemaphoreType.DMA((2,)),
            pltpu.SemaphoreType.DMA((4,)),
        ],
        compiler_params=pltpu.CompilerParams(
            collective_id=0,
            vmem_limit_bytes=60 * 1024 * 1024,
        ),
    )
    return call(A, B.astype(jnp.bfloat16))
